# baseline (speedup 1.0000x reference)
import functools

import jax
import jax.numpy as jnp
from jax import lax
from jax.experimental import pallas as pl
from jax.experimental.pallas import tpu as pltpu
from jax.experimental.pallas import tpu_sc as plsc

_LANES = 16
_G = 2
_NBUF = 2


@functools.lru_cache(maxsize=None)
def _build(B, L, V, D, C):
  assert D % _LANES == 0
  KD = D // _LANES
  NC, NS = 2, 16
  NW = NC * NS
  assert B % NW == 0
  BPW = B // NW
  NG = BPW // _G
  assert BPW % (_G * _NBUF) == 0
  inv_l = 1.0 / L
  UNROLL = 25
  assert L % UNROLL == 0
  GL = _G * L

  mesh = plsc.VectorSubcoreMesh(core_axis_name="c", subcore_axis_name="s")

  @functools.partial(
      pl.kernel,
      out_type=jax.ShapeDtypeStruct((B, _LANES), jnp.float32),
      mesh=mesh,
      compiler_params=pltpu.CompilerParams(use_tc_tiling_on_sc=False),
      scratch_types=[
          pltpu.VMEM((BPW * L,), jnp.int32),
          pltpu.VMEM((GL, D), jnp.float32),
          pltpu.VMEM((GL, D), jnp.float32),
          pltpu.VMEM((D, _LANES), jnp.float32),
          pltpu.VMEM((_LANES,), jnp.float32),
          pltpu.VMEM((BPW, _LANES), jnp.float32),
          pltpu.SemaphoreType.DMA,
          pltpu.SemaphoreType.DMA,
      ],
  )
  def sc_kernel(text_hbm, table_hbm, fcwt_hbm, fcb_hbm, out_hbm,
                idx_v, rows0_v, rows1_v, fcwt_v, fcb_v, out_v, sem0, sem1):
    wid = lax.axis_index("s") * NC + lax.axis_index("c")
    base = wid * BPW
    bufs = (rows0_v, rows1_v)
    sems = (sem0, sem1)

    pltpu.sync_copy(fcwt_hbm, fcwt_v)
    pltpu.sync_copy(fcb_hbm, fcb_v)
    pltpu.sync_copy(text_hbm.at[pl.ds(base * L, BPW * L)], idx_v)
    fcb_vec = fcb_v[pl.ds(0, _LANES)]

    def fire(g, buf, sem):
      pltpu.async_copy(table_hbm.at[idx_v.at[pl.ds(g * GL, GL)]], buf, sem)

    def drain(buf, sem):
      pltpu.make_async_copy(table_hbm.at[pl.ds(0, GL)], buf, sem).wait()

    def process(g, buf):
      zero = jnp.zeros((_LANES,), jnp.float32)
      for i in range(_G):
        r = g * _G + i

        def red_body(t, accs, i=i):
          ib = i * L + t * UNROLL
          accs = list(accs)
          for u in range(UNROLL):
            for k in range(KD):
              accs[k] = accs[k] + buf[ib + u, pl.ds(k * _LANES, _LANES)]
          return tuple(accs)

        accs = lax.fori_loop(0, L // UNROLL, red_body, (zero,) * KD)
        pooled = [a * inv_l for a in accs]

        parts = [fcb_vec, zero, zero, zero]
        for d in range(D):
          parts[d % 4] = parts[d % 4] + (
              pooled[d // _LANES][d % _LANES] * fcwt_v[d, pl.ds(0, _LANES)])
        out_row = (parts[0] + parts[1]) + (parts[2] + parts[3])
        out_v[r, pl.ds(0, _LANES)] = out_row

    fire(0, bufs[0], sems[0])

    def body(h, carry):
      for b in range(_NBUF):
        g = h * _NBUF + b
        nxt = g + _NBUF - 1
        pb = (b - 1) % _NBUF

        @pl.when(nxt < NG)
        def _():
          fire(nxt, bufs[pb], sems[pb])

        drain(bufs[b], sems[b])
        process(g, bufs[b])
      return carry

    lax.fori_loop(0, NG // _NBUF, body, 0)
    pltpu.sync_copy(out_v, out_hbm.at[pl.ds(base, BPW)])

  return sc_kernel


def kernel(text, emb_table, fc_w, fc_b):
  B, L = text.shape
  V, D = emb_table.shape
  C = fc_w.shape[0]
  text = text.astype(jnp.int32).reshape(-1)
  fcwt = jnp.zeros((D, _LANES), jnp.float32).at[:, :C].set(fc_w.T)
  fcb_pad = jnp.zeros((_LANES,), jnp.float32).at[:C].set(fc_b)
  out = _build(B, L, V, D, C)(text, emb_table, fcwt, fcb_pad)
  return out[:, :C]


# speedup vs baseline: 1.0050x; 1.0050x over previous
import functools

import jax
import jax.numpy as jnp
from jax import lax
from jax.experimental import pallas as pl
from jax.experimental.pallas import tpu as pltpu
from jax.experimental.pallas import tpu_sc as plsc

_LANES = 16
_SEG = 25
_NBUF = 8


@functools.lru_cache(maxsize=None)
def _build(B, L, V, D, C):
  assert D % _LANES == 0
  KD = D // _LANES
  NC, NS = 2, 16
  NW = NC * NS
  assert B % NW == 0
  BPW = B // NW
  assert BPW % _NBUF == 0
  assert L % _SEG == 0
  NSEG = L // _SEG
  inv_l = 1.0 / L

  mesh = plsc.VectorSubcoreMesh(core_axis_name="c", subcore_axis_name="s")

  @functools.partial(
      pl.kernel,
      out_type=jax.ShapeDtypeStruct((B, _LANES), jnp.float32),
      mesh=mesh,
      compiler_params=pltpu.CompilerParams(use_tc_tiling_on_sc=False),
      scratch_types=[
          pltpu.VMEM((BPW * NSEG, _SEG), jnp.int32),
          pltpu.VMEM((_NBUF * _SEG, D), jnp.float32),
          pltpu.VMEM((D, _LANES), jnp.float32),
          pltpu.VMEM((_LANES,), jnp.float32),
          pltpu.VMEM((BPW, _LANES), jnp.float32),
      ] + [pltpu.SemaphoreType.DMA] * _NBUF,
  )
  def sc_kernel(text_hbm, table_hbm, fcwt_hbm, fcb_hbm, out_hbm,
                idx_v, acc_v, fcwt_v, fcb_v, out_v, *sems):
    wid = lax.axis_index("s") * NC + lax.axis_index("c")
    base = wid * BPW

    pltpu.sync_copy(fcwt_hbm, fcwt_v)
    pltpu.sync_copy(fcb_hbm, fcb_v)
    pltpu.sync_copy(text_hbm.at[pl.ds(base * NSEG, BPW * NSEG)], idx_v)
    fcb_vec = fcb_v[pl.ds(0, _LANES)]
    zero = jnp.zeros((_LANES,), jnp.float32)

    def z_body(i, carry):
      for k in range(KD):
        acc_v[i, pl.ds(k * _LANES, _LANES)] = zero
      return carry

    lax.fori_loop(0, _NBUF * _SEG, z_body, 0)

    def fire(r, b):
      dst = acc_v.at[pl.ds(b * _SEG, _SEG)]
      for j in range(NSEG):
        pltpu.async_copy(table_hbm.at[idx_v.at[r * NSEG + j]], dst,
                         sems[b], add=True)

    def drain(b):
      dst = acc_v.at[pl.ds(b * _SEG, _SEG)]
      for _ in range(NSEG):
        pltpu.make_async_copy(table_hbm.at[pl.ds(0, _SEG)], dst,
                              sems[b]).wait()

    def process(r, b):
      off = b * _SEG
      accs = [zero] * KD
      for i in range(_SEG):
        for k in range(KD):
          accs[k] = accs[k] + acc_v[off + i, pl.ds(k * _LANES, _LANES)]
      for i in range(_SEG):
        for k in range(KD):
          acc_v[off + i, pl.ds(k * _LANES, _LANES)] = zero
      pooled = [a * inv_l for a in accs]

      parts = [fcb_vec, zero, zero, zero]
      for d in range(D):
        parts[d % 4] = parts[d % 4] + (
            pooled[d // _LANES][d % _LANES] * fcwt_v[d, pl.ds(0, _LANES)])
      out_row = (parts[0] + parts[1]) + (parts[2] + parts[3])
      out_v[r, pl.ds(0, _LANES)] = out_row

    for b in range(_NBUF - 1):
      fire(b, b)

    def body(h, carry):
      for b in range(_NBUF):
        r = h * _NBUF + b
        nxt = r + _NBUF - 1
        pb = (b - 1) % _NBUF

        @pl.when(nxt < BPW)
        def _():
          fire(nxt, pb)

        drain(b)
        process(r, b)
      return carry

    lax.fori_loop(0, BPW // _NBUF, body, 0)
    pltpu.sync_copy(out_v, out_hbm.at[pl.ds(base, BPW)])

  return sc_kernel


def kernel(text, emb_table, fc_w, fc_b):
  B, L = text.shape
  V, D = emb_table.shape
  C = fc_w.shape[0]
  text = text.astype(jnp.int32).reshape(B * (L // _SEG), _SEG)
  fcwt = jnp.zeros((D, _LANES), jnp.float32).at[:, :C].set(fc_w.T)
  fcb_pad = jnp.zeros((_LANES,), jnp.float32).at[:C].set(fc_b)
  out = _build(B, L, V, D, C)(text, emb_table, fcwt, fcb_pad)
  return out[:, :C]
